# baseline (speedup 1.0000x reference)
import functools

import jax
import jax.numpy as jnp
from jax import lax
from jax.experimental import pallas as pl
from jax.experimental.pallas import tpu as pltpu


def _gather_sum_kernel(tok_ref, lut_ref, bc_ref, o_ref, *, n_pos, vocab):
    tb = tok_ref.shape[0]
    tok = tok_ref[...]
    v_iota = lax.broadcasted_iota(jnp.int32, (tb, vocab), 1)
    acc = None
    for l in range(n_pos):
        onehot = jnp.where(tok[:, l:l + 1] == v_iota,
                           jnp.float32(1), jnp.float32(0)
                           ).astype(jnp.bfloat16)
        d = jnp.dot(onehot, lut_ref[l * vocab:(l + 1) * vocab, :],
                    preferred_element_type=jnp.float32)
        acc = d if acc is None else acc + d
    o_ref[...] = acc + bc_ref[...]


@functools.partial(jax.jit, static_argnames=())
def kernel(tokens, lut, bc):
    B, L = tokens.shape
    _, V, C = lut.shape
    lut_flat = lut.astype(jnp.bfloat16).reshape(L * V, C)
    bc_f = bc.astype(jnp.float32)

    tb = 1024
    n_tiles = -(-B // tb)
    b_pad = n_tiles * tb
    tok_p = tokens.astype(jnp.int32)
    if b_pad != B:
        tok_p = jnp.pad(tok_p, ((0, b_pad - B), (0, 0)))

    cost = pl.CostEstimate(
        flops=2 * b_pad * L * V * C, transcendentals=0,
        bytes_accessed=4 * b_pad * L + 2 * lut_flat.size + 4 * C + 4 * b_pad * C)
    out = pl.pallas_call(
        functools.partial(_gather_sum_kernel, n_pos=L, vocab=V),
        out_shape=jax.ShapeDtypeStruct((b_pad, C), jnp.float32),
        grid=(n_tiles,),
        in_specs=[
            pl.BlockSpec((tb, L), lambda i: (i, 0)),
            pl.BlockSpec((L * V, C), lambda i: (0, 0)),
            pl.BlockSpec((1, C), lambda i: (0, 0)),
        ],
        out_specs=pl.BlockSpec((tb, C), lambda i: (i, 0)),
        compiler_params=pltpu.CompilerParams(
            dimension_semantics=("parallel",)),
        cost_estimate=cost,
    )(tok_p, lut_flat, bc_f)
    return out[:B, :C]


# speedup vs baseline: 1.4527x; 1.4527x over previous
import functools

import numpy as np

import jax
import jax.numpy as jnp
from jax import lax
from jax.experimental import pallas as pl
from jax.experimental.pallas import tpu as pltpu
from jax.experimental.shard_map import shard_map
from jax.sharding import Mesh, PartitionSpec as P


def _onehot_lut_kernel(tok_ref, lut_ref, bc_ref, o_ref, *, n_pos, vocab):
    tb = tok_ref.shape[0]
    tok = tok_ref[...]
    v_iota = lax.broadcasted_iota(jnp.int32, (tb, vocab), 1)
    acc = None
    for l in range(n_pos):
        onehot = jnp.where(tok[:, l:l + 1] == v_iota,
                           jnp.float32(1), jnp.float32(0))
        d = jnp.dot(onehot, lut_ref[l * vocab:(l + 1) * vocab, :],
                    preferred_element_type=jnp.float32)
        acc = d if acc is None else acc + d
    o_ref[...] = acc + bc_ref[...]


def _single_device(tokens, lut_flat, bc_f, *, n_pos, vocab, n_classes):
    B = tokens.shape[0]
    tb = 2048
    n_tiles = -(-B // tb)
    b_pad = n_tiles * tb
    tok_p = tokens
    if b_pad != B:
        tok_p = jnp.pad(tok_p, ((0, b_pad - B), (0, 0)))
    cost = pl.CostEstimate(
        flops=2 * b_pad * n_pos * vocab * n_classes, transcendentals=0,
        bytes_accessed=4 * b_pad * n_pos + 4 * lut_flat.size
        + 4 * n_classes + 4 * b_pad * n_classes)
    out = pl.pallas_call(
        functools.partial(_onehot_lut_kernel, n_pos=n_pos, vocab=vocab),
        out_shape=jax.ShapeDtypeStruct((b_pad, n_classes), jnp.float32),
        grid=(n_tiles,),
        in_specs=[
            pl.BlockSpec((tb, n_pos), lambda i: (i, 0)),
            pl.BlockSpec((n_pos * vocab, n_classes), lambda i: (0, 0)),
            pl.BlockSpec((1, n_classes), lambda i: (0, 0)),
        ],
        out_specs=pl.BlockSpec((tb, n_classes), lambda i: (i, 0)),
        compiler_params=pltpu.CompilerParams(
            dimension_semantics=("parallel",)),
        cost_estimate=cost,
    )(tok_p, lut_flat, bc_f)
    return out[:B, :]


@jax.jit
def kernel(tokens, lut, bc):
    B, L = tokens.shape
    _, V, C = lut.shape
    lut_flat = lut.reshape(L * V, C).astype(jnp.float32)
    bc_f = bc.astype(jnp.float32)
    tok_p = tokens.astype(jnp.int32)

    impl = functools.partial(_single_device, n_pos=L, vocab=V, n_classes=C)

    devs = jax.devices()
    n_dev = len(devs)
    if n_dev > 1 and B % n_dev == 0:
        mesh = Mesh(np.array(devs), ("b",))
        sharded = shard_map(
            impl, mesh=mesh,
            in_specs=(P("b", None), P(None, None), P(None, None)),
            out_specs=P("b", None), check_rep=False)
        return sharded(tok_p, lut_flat, bc_f)
    return impl(tok_p, lut_flat, bc_f)


# speedup vs baseline: 1.4936x; 1.0282x over previous
import functools

import numpy as np

import jax
import jax.numpy as jnp
from jax import lax
from jax.experimental import pallas as pl
from jax.experimental.pallas import tpu as pltpu
from jax.experimental.shard_map import shard_map
from jax.sharding import Mesh, PartitionSpec as P


def _onehot_lut_kernel(tok_ref, lut_ref, bc_ref, o_ref, *, n_pos, vocab):
    tb = tok_ref.shape[0]
    tok = tok_ref[...].astype(jnp.int32)
    v_iota = lax.broadcasted_iota(jnp.int32, (tb, vocab), 1)
    acc = None
    for l in range(n_pos):
        onehot = jnp.where(tok[:, l:l + 1] == v_iota,
                           jnp.float32(1), jnp.float32(0))
        d = jnp.dot(onehot, lut_ref[l * vocab:(l + 1) * vocab, :],
                    preferred_element_type=jnp.float32)
        acc = d if acc is None else acc + d
    o_ref[...] = acc + bc_ref[...]


def _single_device(tokens, lut_bf, bc_f, *, n_pos, vocab, n_classes):
    B = tokens.shape[0]
    lut_flat = lut_bf.astype(jnp.float32)
    tb = 2048
    n_tiles = -(-B // tb)
    b_pad = n_tiles * tb
    tok_p = tokens
    if b_pad != B:
        tok_p = jnp.pad(tok_p, ((0, b_pad - B), (0, 0)))
    cost = pl.CostEstimate(
        flops=2 * b_pad * n_pos * vocab * n_classes, transcendentals=0,
        bytes_accessed=4 * b_pad * n_pos + 4 * lut_flat.size
        + 4 * n_classes + 4 * b_pad * n_classes)
    out = pl.pallas_call(
        functools.partial(_onehot_lut_kernel, n_pos=n_pos, vocab=vocab),
        out_shape=jax.ShapeDtypeStruct((b_pad, n_classes), jnp.float32),
        grid=(n_tiles,),
        in_specs=[
            pl.BlockSpec((tb, n_pos), lambda i: (i, 0)),
            pl.BlockSpec((n_pos * vocab, n_classes), lambda i: (0, 0)),
            pl.BlockSpec((1, n_classes), lambda i: (0, 0)),
        ],
        out_specs=pl.BlockSpec((tb, n_classes), lambda i: (i, 0)),
        compiler_params=pltpu.CompilerParams(
            dimension_semantics=("parallel",)),
        cost_estimate=cost,
    )(tok_p, lut_flat, bc_f)
    return out[:B, :]


@jax.jit
def kernel(tokens, lut, bc):
    B, L = tokens.shape
    _, V, C = lut.shape
    lut_bf = lut.reshape(L * V, C).astype(jnp.bfloat16)
    bc_f = bc.astype(jnp.float32)
    tok_p = tokens.astype(jnp.int16)

    impl = functools.partial(_single_device, n_pos=L, vocab=V, n_classes=C)

    devs = jax.devices()
    n_dev = len(devs)
    if n_dev > 1 and B % n_dev == 0:
        mesh = Mesh(np.array(devs), ("b",))
        sharded = shard_map(
            impl, mesh=mesh,
            in_specs=(P("b", None), P(None, None), P(None, None)),
            out_specs=P("b", None), check_rep=False)
        return sharded(tok_p, lut_bf, bc_f)
    return impl(tok_p, lut_bf, bc_f)


# speedup vs baseline: 2.0880x; 1.3980x over previous
import functools

import numpy as np

import jax
import jax.numpy as jnp
from jax import lax
from jax.experimental import pallas as pl
from jax.experimental.pallas import tpu as pltpu
from jax.experimental.shard_map import shard_map
from jax.sharding import Mesh, PartitionSpec as P


def _onehot_lut_kernel(tok_ref, lut_ref, bc_ref, o_ref, *, n_pos, vocab):
    tb = tok_ref.shape[0]
    tok = tok_ref[...].astype(jnp.int32)
    v_iota = lax.broadcasted_iota(jnp.int32, (tb, vocab), 1)
    o_ref[...] = jnp.broadcast_to(bc_ref[...], o_ref.shape)
    for l in range(n_pos):
        onehot = jnp.where(tok[:, l:l + 1] == v_iota,
                           jnp.float32(1), jnp.float32(0))
        o_ref[...] += jnp.dot(onehot, lut_ref[l * vocab:(l + 1) * vocab, :],
                              preferred_element_type=jnp.float32)


def _single_device(tokens, lut_bf, bc_f, *, n_pos, vocab, n_classes):
    B = tokens.shape[0]
    lut_flat = lut_bf.astype(jnp.float32)
    tb = 2048
    n_tiles = -(-B // tb)
    b_pad = n_tiles * tb
    tok_p = tokens
    if b_pad != B:
        tok_p = jnp.pad(tok_p, ((0, b_pad - B), (0, 0)))
    cost = pl.CostEstimate(
        flops=2 * b_pad * n_pos * vocab * n_classes, transcendentals=0,
        bytes_accessed=4 * b_pad * n_pos + 4 * lut_flat.size
        + 4 * n_classes + 4 * b_pad * n_classes)
    out = pl.pallas_call(
        functools.partial(_onehot_lut_kernel, n_pos=n_pos, vocab=vocab),
        out_shape=jax.ShapeDtypeStruct((b_pad, n_classes), jnp.float32),
        grid=(n_tiles,),
        in_specs=[
            pl.BlockSpec((tb, n_pos), lambda i: (i, 0)),
            pl.BlockSpec((n_pos * vocab, n_classes), lambda i: (0, 0)),
            pl.BlockSpec((1, n_classes), lambda i: (0, 0)),
        ],
        out_specs=pl.BlockSpec((tb, n_classes), lambda i: (i, 0)),
        compiler_params=pltpu.CompilerParams(
            dimension_semantics=("parallel",)),
        cost_estimate=cost,
    )(tok_p, lut_flat, bc_f)
    return out[:B, :]


@jax.jit
def kernel(tokens, lut, bc):
    B, L = tokens.shape
    _, V, C = lut.shape
    lut_bf = lut.reshape(L * V, C).astype(jnp.bfloat16)
    bc_f = bc.astype(jnp.float32)
    tok_p = tokens.astype(jnp.int16)

    impl = functools.partial(_single_device, n_pos=L, vocab=V, n_classes=C)

    devs = jax.devices()
    n_dev = len(devs)
    if n_dev > 1 and B % n_dev == 0:
        shard_rows = B // n_dev
        mesh = Mesh(np.array(devs), ("b",))

        def _shard_fn(tok_all, lut_s, bc_s):
            i = lax.axis_index("b")
            tok_local = lax.dynamic_slice_in_dim(
                tok_all, i * shard_rows, shard_rows, axis=0)
            return impl(tok_local, lut_s, bc_s)

        sharded = shard_map(
            _shard_fn, mesh=mesh,
            in_specs=(P(None, None), P(None, None), P(None, None)),
            out_specs=P("b", None), check_rep=False)
        return sharded(tok_p, lut_bf, bc_f)
    return impl(tok_p, lut_bf, bc_f)


# speedup vs baseline: 2.1025x; 1.0069x over previous
import functools

import numpy as np

import jax
import jax.numpy as jnp
from jax import lax
from jax.experimental import pallas as pl
from jax.experimental.pallas import tpu as pltpu
from jax.experimental.shard_map import shard_map
from jax.sharding import Mesh, PartitionSpec as P


def _onehot_lut_kernel(tok_ref, lut_ref, bc_ref, o_ref, *, n_pos, vocab):
    tb = tok_ref.shape[0]
    tok = tok_ref[...].astype(jnp.int32)
    v_iota = lax.broadcasted_iota(jnp.int32, (tb, vocab), 1)
    o_ref[...] = jnp.broadcast_to(bc_ref[...], o_ref.shape)
    for l in range(n_pos):
        onehot = jnp.where(tok[:, l:l + 1] == v_iota,
                           jnp.float32(1), jnp.float32(0))
        o_ref[...] += jnp.dot(onehot, lut_ref[l * vocab:(l + 1) * vocab, :],
                              preferred_element_type=jnp.float32)


def _single_device(tokens, lut_bf, bc_f, *, n_pos, vocab, n_classes):
    B = tokens.shape[0]
    lut_flat = lut_bf.astype(jnp.float32)
    tb = 4096
    n_tiles = -(-B // tb)
    b_pad = n_tiles * tb
    tok_p = tokens
    if b_pad != B:
        tok_p = jnp.pad(tok_p, ((0, b_pad - B), (0, 0)))
    cost = pl.CostEstimate(
        flops=2 * b_pad * n_pos * vocab * n_classes, transcendentals=0,
        bytes_accessed=4 * b_pad * n_pos + 4 * lut_flat.size
        + 4 * n_classes + 4 * b_pad * n_classes)
    out = pl.pallas_call(
        functools.partial(_onehot_lut_kernel, n_pos=n_pos, vocab=vocab),
        out_shape=jax.ShapeDtypeStruct((b_pad, n_classes), jnp.float32),
        grid=(n_tiles,),
        in_specs=[
            pl.BlockSpec((tb, n_pos), lambda i: (i, 0)),
            pl.BlockSpec((n_pos * vocab, n_classes), lambda i: (0, 0)),
            pl.BlockSpec((1, n_classes), lambda i: (0, 0)),
        ],
        out_specs=pl.BlockSpec((tb, n_classes), lambda i: (i, 0)),
        compiler_params=pltpu.CompilerParams(
            dimension_semantics=("parallel",)),
        cost_estimate=cost,
    )(tok_p, lut_flat, bc_f)
    return out[:B, :]


@jax.jit
def kernel(tokens, lut, bc):
    B, L = tokens.shape
    _, V, C = lut.shape
    lut_bf = lut.reshape(L * V, C).astype(jnp.bfloat16)
    bc_f = bc.astype(jnp.float32)
    tok_p = tokens.astype(jnp.int16)

    impl = functools.partial(_single_device, n_pos=L, vocab=V, n_classes=C)

    devs = jax.devices()
    n_dev = len(devs)
    if n_dev > 1 and B % n_dev == 0:
        shard_rows = B // n_dev
        mesh = Mesh(np.array(devs), ("b",))

        def _shard_fn(tok_all, lut_s, bc_s):
            i = lax.axis_index("b")
            tok_local = lax.dynamic_slice_in_dim(
                tok_all, i * shard_rows, shard_rows, axis=0)
            return impl(tok_local, lut_s, bc_s)

        sharded = shard_map(
            _shard_fn, mesh=mesh,
            in_specs=(P(None, None), P(None, None), P(None, None)),
            out_specs=P("b", None), check_rep=False)
        return sharded(tok_p, lut_bf, bc_f)
    return impl(tok_p, lut_bf, bc_f)
